# baseline (speedup 1.0000x reference)
import functools

import jax
import jax.numpy as jnp
from jax import lax
from jax.experimental import pallas as pl
from jax.experimental.pallas import tpu as pltpu
from jax.experimental.pallas import tpu_sc as plsc

_N = 18432
_D = 64
_V = 8192

_TM = 512
_CB = 512
_LANES = 128

_NC = 2
_NS = 16
_NW = _NC * _NS
_BW = _N // _NW
_IG = 96
_NG = _BW // _IG

_EPS = 1e-12
_INT_MAX = 2**31 - 1


def _rowsum64(a2):
    acc = a2[:, 0:8]
    for k in range(1, 8):
        acc = acc + a2[:, 8 * k:8 * k + 8]
    p01 = acc[:, 0:1] + acc[:, 1:2]
    p23 = acc[:, 2:3] + acc[:, 3:4]
    p45 = acc[:, 4:5] + acc[:, 5:6]
    p67 = acc[:, 6:7] + acc[:, 7:8]
    return (p01 + p23) + (p45 + p67)


def _l2norm_rows(a):
    s = _rowsum64(a * a)
    n = jnp.sqrt(s)
    n = n + (s - n * n) / (2.0 * n)
    n = jnp.maximum(n, _EPS)
    q = a / n
    return q + (a - q * n) / n


def _lane_argmin(bd, bi):
    m = jnp.min(bd, axis=1, keepdims=True)
    cand = jnp.where(bd == m, bi, jnp.int32(_INT_MAX))
    return m, jnp.min(cand, axis=1, keepdims=True)


def _argmin_body(x_ref, w_ref, out_ref, bd_ref, bi_ref, m1_ref, i1_ref):
    c = pl.program_id(1)

    xn = _l2norm_rows(x_ref[...])
    wn = _l2norm_rows(w_ref[...])

    t = lax.dot_general(xn.astype(jnp.bfloat16), wn.astype(jnp.bfloat16),
                        (((1,), (1,)), ((), ())),
                        preferred_element_type=jnp.float32)
    d = 2.0 - 2.0 * t

    half = pl.num_programs(1) // 2

    @pl.when((c == 0) | (c == half))
    def _():
        bd_ref[...] = jnp.full((_TM, _LANES), jnp.inf, jnp.float32)
        bi_ref[...] = jnp.zeros((_TM, _LANES), jnp.int32)

    lane = lax.broadcasted_iota(jnp.int32, (_TM, _LANES), 1)
    for k in range(_CB // _LANES):
        cols = c * _CB + k * _LANES + lane
        sk = d[:, k * _LANES:(k + 1) * _LANES]
        bd = bd_ref[...]
        upd = sk < bd
        bd_ref[...] = jnp.where(upd, sk, bd)
        bi_ref[...] = jnp.where(upd, cols, bi_ref[...])

    @pl.when(c == half - 1)
    def _():
        m1, i1 = _lane_argmin(bd_ref[...], bi_ref[...])
        m1_ref[...] = m1.astype(jnp.bfloat16).astype(jnp.float32)
        i1_ref[...] = i1

    @pl.when(c == pl.num_programs(1) - 1)
    def _():
        m2, i2 = _lane_argmin(bd_ref[...], bi_ref[...])
        take2 = m2 < m1_ref[...]
        out_ref[...] = jnp.where(take2, i2, i1_ref[...])


def _argmin_indices(x, W):
    return pl.pallas_call(
        _argmin_body,
        grid=(_N // _TM, _V // _CB),
        in_specs=[
            pl.BlockSpec((_TM, _D), lambda t, c: (t, 0)),
            pl.BlockSpec((_CB, _D), lambda t, c: (c, 0)),
        ],
        out_specs=pl.BlockSpec((_TM, 1), lambda t, c: (t, 0)),
        out_shape=jax.ShapeDtypeStruct((_N, 1), jnp.int32),
        scratch_shapes=[
            pltpu.VMEM((_TM, _LANES), jnp.float32),
            pltpu.VMEM((_TM, _LANES), jnp.int32),
            pltpu.VMEM((_TM, 1), jnp.float32),
            pltpu.VMEM((_TM, 1), jnp.int32),
        ],
        compiler_params=pltpu.CompilerParams(
            dimension_semantics=("parallel", "arbitrary")),
    )(x, W)


def _sc_gather_body(table_hbm, idx_hbm, out_hbm, idx_v, rows_v, sem):
    wid = lax.axis_index("s") * _NC + lax.axis_index("c")
    base = wid * _BW
    pltpu.sync_copy(idx_hbm.at[wid], idx_v)
    copies = []
    for g in range(_NG):
        copies.append(pltpu.async_copy(
            table_hbm.at[idx_v.at[g]],
            rows_v.at[pl.ds(g * _IG, _IG)],
            sem))
    for cp in copies:
        cp.wait()
    pltpu.sync_copy(rows_v, out_hbm.at[pl.ds(base, _BW)])


def _sc_gather(W, idx_groups):
    mesh = plsc.VectorSubcoreMesh(core_axis_name="c", subcore_axis_name="s")
    k = functools.partial(
        pl.kernel,
        out_type=jax.ShapeDtypeStruct((_N, _D), jnp.float32),
        scratch_types=[
            pltpu.VMEM((_NG, _IG), jnp.int32),
            pltpu.VMEM((_BW, _D), jnp.float32),
            pltpu.SemaphoreType.DMA,
        ],
        mesh=mesh,
        compiler_params=pltpu.CompilerParams(use_tc_tiling_on_sc=False),
    )(_sc_gather_body)
    return k(W, idx_groups)


def _st_loss_body(x_ref, q_ref, st_ref, loss_ref, acc_ref):
    i = pl.program_id(0)
    x = x_ref[...]
    q = q_ref[...]
    dqx = q - x
    st_ref[...] = x + dqx
    s = jnp.sum(dqx * dqx)

    @pl.when(i == 0)
    def _():
        acc_ref[0, 0] = 0.0

    acc_ref[0, 0] += s

    @pl.when(i == pl.num_programs(0) - 1)
    def _():
        loss_ref[...] = jnp.full(
            (1, 1), acc_ref[0, 0] * (1.0 / float(_N * _D)), jnp.float32)


def _st_loss(x, q):
    return pl.pallas_call(
        _st_loss_body,
        grid=(_N // _TM,),
        in_specs=[
            pl.BlockSpec((_TM, _D), lambda i: (i, 0)),
            pl.BlockSpec((_TM, _D), lambda i: (i, 0)),
        ],
        out_specs=[
            pl.BlockSpec((_TM, _D), lambda i: (i, 0)),
            pl.BlockSpec((1, 1), lambda i: (0, 0)),
        ],
        out_shape=[
            jax.ShapeDtypeStruct((_N, _D), jnp.float32),
            jax.ShapeDtypeStruct((1, 1), jnp.float32),
        ],
        scratch_shapes=[pltpu.SMEM((1, 1), jnp.float32)],
        compiler_params=pltpu.CompilerParams(
            dimension_semantics=("arbitrary",)),
    )(x, q)


def kernel(inputs, W):
    x = inputs.reshape(-1, _D)
    idx2d = _argmin_indices(x, W)
    idx_groups = idx2d.reshape(_NW, _NG, _IG)
    q = _sc_gather(W, idx_groups)
    st, loss = _st_loss(x, q)
    return st.reshape(inputs.shape), loss.reshape(()), idx2d


# speedup vs baseline: 2.3345x; 2.3345x over previous
import functools

import jax
import jax.numpy as jnp
from jax import lax
from jax.experimental import pallas as pl
from jax.experimental.pallas import tpu as pltpu
from jax.experimental.pallas import tpu_sc as plsc

_N = 18432
_D = 64
_V = 8192

_TM = 512
_CB = 512
_LANES = 128

_NC = 2
_NS = 16
_NW = _NC * _NS
_BW = _N // _NW
_IG = 96
_NG = _BW // _IG

_EPS = 1e-12
_INT_MAX = 2**31 - 1


def _rowsum64(a2):
    acc = a2[:, 0:8]
    for k in range(1, 8):
        acc = acc + a2[:, 8 * k:8 * k + 8]
    p01 = acc[:, 0:1] + acc[:, 1:2]
    p23 = acc[:, 2:3] + acc[:, 3:4]
    p45 = acc[:, 4:5] + acc[:, 5:6]
    p67 = acc[:, 6:7] + acc[:, 7:8]
    return (p01 + p23) + (p45 + p67)


def _l2norm_rows(a):
    s = _rowsum64(a * a)
    n = jnp.sqrt(s)
    n = n + (s - n * n) / (2.0 * n)
    n = jnp.maximum(n, _EPS)
    q = a / n
    return q + (a - q * n) / n


def _lane_argmin(bd, bi):
    m = jnp.min(bd, axis=1, keepdims=True)
    cand = jnp.where(bd == m, bi, jnp.int32(_INT_MAX))
    return m, jnp.min(cand, axis=1, keepdims=True)


def _normalize_body(a_ref, o_ref):
    o_ref[...] = _l2norm_rows(a_ref[...]).astype(jnp.bfloat16)


def _normalize_bf16(a, tile):
    return pl.pallas_call(
        _normalize_body,
        grid=(a.shape[0] // tile,),
        in_specs=[pl.BlockSpec((tile, _D), lambda i: (i, 0))],
        out_specs=pl.BlockSpec((tile, _D), lambda i: (i, 0)),
        out_shape=jax.ShapeDtypeStruct(a.shape, jnp.bfloat16),
        compiler_params=pltpu.CompilerParams(
            dimension_semantics=("parallel",)),
    )(a)


def _argmin_body(x_ref, w_ref, out_ref, bd_ref, bi_ref, m1_ref, i1_ref):
    c = pl.program_id(1)

    t = lax.dot_general(x_ref[...], w_ref[...],
                        (((1,), (1,)), ((), ())),
                        preferred_element_type=jnp.float32)
    d = 2.0 - 2.0 * t

    half = pl.num_programs(1) // 2

    @pl.when((c == 0) | (c == half))
    def _():
        bd_ref[...] = jnp.full((_TM, _LANES), jnp.inf, jnp.float32)
        bi_ref[...] = jnp.zeros((_TM, _LANES), jnp.int32)

    lane = lax.broadcasted_iota(jnp.int32, (_TM, _LANES), 1)
    for k in range(_CB // _LANES):
        cols = c * _CB + k * _LANES + lane
        sk = d[:, k * _LANES:(k + 1) * _LANES]
        bd = bd_ref[...]
        upd = sk < bd
        bd_ref[...] = jnp.where(upd, sk, bd)
        bi_ref[...] = jnp.where(upd, cols, bi_ref[...])

    @pl.when(c == half - 1)
    def _():
        m1, i1 = _lane_argmin(bd_ref[...], bi_ref[...])
        m1_ref[...] = m1.astype(jnp.bfloat16).astype(jnp.float32)
        i1_ref[...] = i1

    @pl.when(c == pl.num_programs(1) - 1)
    def _():
        m2, i2 = _lane_argmin(bd_ref[...], bi_ref[...])
        take2 = m2 < m1_ref[...]
        out_ref[...] = jnp.where(take2, i2, i1_ref[...])


def _argmin_indices(x, W):
    return pl.pallas_call(
        _argmin_body,
        grid=(_N // _TM, _V // _CB),
        in_specs=[
            pl.BlockSpec((_TM, _D), lambda t, c: (t, 0)),
            pl.BlockSpec((_CB, _D), lambda t, c: (c, 0)),
        ],
        out_specs=pl.BlockSpec((_TM, 1), lambda t, c: (t, 0)),
        out_shape=jax.ShapeDtypeStruct((_N, 1), jnp.int32),
        scratch_shapes=[
            pltpu.VMEM((_TM, _LANES), jnp.float32),
            pltpu.VMEM((_TM, _LANES), jnp.int32),
            pltpu.VMEM((_TM, 1), jnp.float32),
            pltpu.VMEM((_TM, 1), jnp.int32),
        ],
        compiler_params=pltpu.CompilerParams(
            dimension_semantics=("parallel", "arbitrary")),
    )(x, W)


def _sc_gather_body(table_hbm, idx_hbm, out_hbm, idx_v, rows_v, sem):
    wid = lax.axis_index("s") * _NC + lax.axis_index("c")
    base = wid * _BW
    pltpu.sync_copy(idx_hbm.at[wid], idx_v)
    copies = []
    for g in range(_NG):
        copies.append(pltpu.async_copy(
            table_hbm.at[idx_v.at[g]],
            rows_v.at[pl.ds(g * _IG, _IG)],
            sem))
    for cp in copies:
        cp.wait()
    pltpu.sync_copy(rows_v, out_hbm.at[pl.ds(base, _BW)])


def _sc_gather(W, idx_groups):
    mesh = plsc.VectorSubcoreMesh(core_axis_name="c", subcore_axis_name="s")
    k = functools.partial(
        pl.kernel,
        out_type=jax.ShapeDtypeStruct((_N, _D), jnp.float32),
        scratch_types=[
            pltpu.VMEM((_NG, _IG), jnp.int32),
            pltpu.VMEM((_BW, _D), jnp.float32),
            pltpu.SemaphoreType.DMA,
        ],
        mesh=mesh,
        compiler_params=pltpu.CompilerParams(use_tc_tiling_on_sc=False),
    )(_sc_gather_body)
    return k(W, idx_groups)


def _st_loss_body(x_ref, q_ref, st_ref, loss_ref, acc_ref):
    i = pl.program_id(0)
    x = x_ref[...]
    q = q_ref[...]
    dqx = q - x
    st_ref[...] = x + dqx
    s = jnp.sum(dqx * dqx)

    @pl.when(i == 0)
    def _():
        acc_ref[0, 0] = 0.0

    acc_ref[0, 0] += s

    @pl.when(i == pl.num_programs(0) - 1)
    def _():
        loss_ref[...] = jnp.full(
            (1, 1), acc_ref[0, 0] * (1.0 / float(_N * _D)), jnp.float32)


def _st_loss(x, q):
    return pl.pallas_call(
        _st_loss_body,
        grid=(_N // _TM,),
        in_specs=[
            pl.BlockSpec((_TM, _D), lambda i: (i, 0)),
            pl.BlockSpec((_TM, _D), lambda i: (i, 0)),
        ],
        out_specs=[
            pl.BlockSpec((_TM, _D), lambda i: (i, 0)),
            pl.BlockSpec((1, 1), lambda i: (0, 0)),
        ],
        out_shape=[
            jax.ShapeDtypeStruct((_N, _D), jnp.float32),
            jax.ShapeDtypeStruct((1, 1), jnp.float32),
        ],
        scratch_shapes=[pltpu.SMEM((1, 1), jnp.float32)],
        compiler_params=pltpu.CompilerParams(
            dimension_semantics=("arbitrary",)),
    )(x, q)


def kernel(inputs, W):
    x = inputs.reshape(-1, _D)
    xn = _normalize_bf16(x, _TM)
    wn = _normalize_bf16(W, _CB)
    idx2d = _argmin_indices(xn, wn)
    idx_groups = idx2d.reshape(_NW, _NG, _IG)
    q = _sc_gather(W, idx_groups)
    st, loss = _st_loss(x, q)
    return st.reshape(inputs.shape), loss.reshape(()), idx2d


# speedup vs baseline: 3.3038x; 1.4152x over previous
import functools

import jax
import jax.numpy as jnp
from jax import lax
from jax.experimental import pallas as pl
from jax.experimental.pallas import tpu as pltpu
from jax.experimental.pallas import tpu_sc as plsc

_N = 18432
_D = 64
_V = 8192

_TM = 1024
_CB = 1024
_LANES = 128

_NC = 2
_NS = 16
_NW = _NC * _NS
_BW = _N // _NW
_IG = 96
_NG = _BW // _IG

_EPS = 1e-12
_INT_MAX = 2**31 - 1


def _rowsum64(a2):
    acc = a2[:, 0:8]
    for k in range(1, 8):
        acc = acc + a2[:, 8 * k:8 * k + 8]
    p01 = acc[:, 0:1] + acc[:, 1:2]
    p23 = acc[:, 2:3] + acc[:, 3:4]
    p45 = acc[:, 4:5] + acc[:, 5:6]
    p67 = acc[:, 6:7] + acc[:, 7:8]
    return (p01 + p23) + (p45 + p67)


def _l2norm_rows(a):
    s = _rowsum64(a * a)
    n = jnp.sqrt(s)
    n = n + (s - n * n) / (2.0 * n)
    n = jnp.maximum(n, _EPS)
    q = a / n
    return q + (a - q * n) / n


def _lane_argmin(bd, bi):
    m = jnp.min(bd, axis=1, keepdims=True)
    cand = jnp.where(bd == m, bi, jnp.int32(_INT_MAX))
    return m, jnp.min(cand, axis=1, keepdims=True)


def _normalize_body(a_ref, o_ref):
    o_ref[...] = _l2norm_rows(a_ref[...]).astype(jnp.bfloat16)


def _normalize_bf16(a, tile):
    return pl.pallas_call(
        _normalize_body,
        grid=(a.shape[0] // tile,),
        in_specs=[pl.BlockSpec((tile, _D), lambda i: (i, 0))],
        out_specs=pl.BlockSpec((tile, _D), lambda i: (i, 0)),
        out_shape=jax.ShapeDtypeStruct(a.shape, jnp.bfloat16),
        compiler_params=pltpu.CompilerParams(
            dimension_semantics=("parallel",)),
    )(a)


def _argmin_body(x_ref, w_ref, out_ref, bd_ref, bi_ref, m1_ref, i1_ref):
    c = pl.program_id(1)

    t = lax.dot_general(x_ref[...], w_ref[...],
                        (((1,), (1,)), ((), ())),
                        preferred_element_type=jnp.float32)
    d = 2.0 - 2.0 * t

    half = pl.num_programs(1) // 2

    @pl.when((c == 0) | (c == half))
    def _():
        bd_ref[...] = jnp.full((_TM, _LANES), jnp.inf, jnp.float32)
        bi_ref[...] = jnp.zeros((_TM, _LANES), jnp.int32)

    lane = lax.broadcasted_iota(jnp.int32, (_TM, _LANES), 1)
    for k in range(_CB // _LANES):
        cols = c * _CB + k * _LANES + lane
        sk = d[:, k * _LANES:(k + 1) * _LANES]
        bd = bd_ref[...]
        upd = sk < bd
        bd_ref[...] = jnp.where(upd, sk, bd)
        bi_ref[...] = jnp.where(upd, cols, bi_ref[...])

    @pl.when(c == half - 1)
    def _():
        m1, i1 = _lane_argmin(bd_ref[...], bi_ref[...])
        m1_ref[...] = m1.astype(jnp.bfloat16).astype(jnp.float32)
        i1_ref[...] = i1

    @pl.when(c == pl.num_programs(1) - 1)
    def _():
        m2, i2 = _lane_argmin(bd_ref[...], bi_ref[...])
        take2 = m2 < m1_ref[...]
        out_ref[...] = jnp.where(take2, i2, i1_ref[...])


def _argmin_indices(x, W):
    return pl.pallas_call(
        _argmin_body,
        grid=(_N // _TM, _V // _CB),
        in_specs=[
            pl.BlockSpec((_TM, _D), lambda t, c: (t, 0)),
            pl.BlockSpec((_CB, _D), lambda t, c: (c, 0)),
        ],
        out_specs=pl.BlockSpec((_TM, 1), lambda t, c: (t, 0)),
        out_shape=jax.ShapeDtypeStruct((_N, 1), jnp.int32),
        scratch_shapes=[
            pltpu.VMEM((_TM, _LANES), jnp.float32),
            pltpu.VMEM((_TM, _LANES), jnp.int32),
            pltpu.VMEM((_TM, 1), jnp.float32),
            pltpu.VMEM((_TM, 1), jnp.int32),
        ],
        compiler_params=pltpu.CompilerParams(
            dimension_semantics=("parallel", "arbitrary")),
    )(x, W)


def _sc_gather_body(table_hbm, idx_hbm, out_hbm, idx_v, rows_v, sem):
    wid = lax.axis_index("s") * _NC + lax.axis_index("c")
    base = wid * _BW
    pltpu.sync_copy(idx_hbm.at[wid], idx_v)
    copies = []
    for g in range(_NG):
        copies.append(pltpu.async_copy(
            table_hbm.at[idx_v.at[g]],
            rows_v.at[pl.ds(g * _IG, _IG)],
            sem))
    for cp in copies:
        cp.wait()
    pltpu.sync_copy(rows_v, out_hbm.at[pl.ds(base, _BW)])


def _sc_gather(W, idx_groups):
    mesh = plsc.VectorSubcoreMesh(core_axis_name="c", subcore_axis_name="s")
    k = functools.partial(
        pl.kernel,
        out_type=jax.ShapeDtypeStruct((_N, _D), jnp.float32),
        scratch_types=[
            pltpu.VMEM((_NG, _IG), jnp.int32),
            pltpu.VMEM((_BW, _D), jnp.float32),
            pltpu.SemaphoreType.DMA,
        ],
        mesh=mesh,
        compiler_params=pltpu.CompilerParams(use_tc_tiling_on_sc=False),
    )(_sc_gather_body)
    return k(W, idx_groups)


def _st_loss_body(x_ref, q_ref, st_ref, loss_ref, acc_ref):
    i = pl.program_id(0)
    x = x_ref[...]
    q = q_ref[...]
    dqx = q - x
    st_ref[...] = x + dqx
    s = jnp.sum(dqx * dqx)

    @pl.when(i == 0)
    def _():
        acc_ref[0, 0] = 0.0

    acc_ref[0, 0] += s

    @pl.when(i == pl.num_programs(0) - 1)
    def _():
        loss_ref[...] = jnp.full(
            (1, 1), acc_ref[0, 0] * (1.0 / float(_N * _D)), jnp.float32)


def _st_loss(x, q):
    return pl.pallas_call(
        _st_loss_body,
        grid=(_N // _TM,),
        in_specs=[
            pl.BlockSpec((_TM, _D), lambda i: (i, 0)),
            pl.BlockSpec((_TM, _D), lambda i: (i, 0)),
        ],
        out_specs=[
            pl.BlockSpec((_TM, _D), lambda i: (i, 0)),
            pl.BlockSpec((1, 1), lambda i: (0, 0)),
        ],
        out_shape=[
            jax.ShapeDtypeStruct((_N, _D), jnp.float32),
            jax.ShapeDtypeStruct((1, 1), jnp.float32),
        ],
        scratch_shapes=[pltpu.SMEM((1, 1), jnp.float32)],
        compiler_params=pltpu.CompilerParams(
            dimension_semantics=("arbitrary",)),
    )(x, q)


def kernel(inputs, W):
    x = inputs.reshape(-1, _D)
    xn = _normalize_bf16(x, _TM)
    wn = _normalize_bf16(W, _CB)
    idx2d = _argmin_indices(xn, wn)
    idx_groups = idx2d.reshape(_NW, _NG, _IG)
    q = _sc_gather(W, idx_groups)
    st, loss = _st_loss(x, q)
    return st.reshape(inputs.shape), loss.reshape(()), idx2d


# speedup vs baseline: 3.3914x; 1.0265x over previous
import functools

import jax
import jax.numpy as jnp
from jax import lax
from jax.experimental import pallas as pl
from jax.experimental.pallas import tpu as pltpu
from jax.experimental.pallas import tpu_sc as plsc

_N = 18432
_D = 64
_V = 8192

_TM = 2048
_CB = 2048
_LANES = 128

_NC = 2
_NS = 16
_NW = _NC * _NS
_BW = _N // _NW
_IG = 96
_NG = _BW // _IG

_EPS = 1e-12
_INT_MAX = 2**31 - 1


def _rowsum64(a2):
    acc = a2[:, 0:8]
    for k in range(1, 8):
        acc = acc + a2[:, 8 * k:8 * k + 8]
    p01 = acc[:, 0:1] + acc[:, 1:2]
    p23 = acc[:, 2:3] + acc[:, 3:4]
    p45 = acc[:, 4:5] + acc[:, 5:6]
    p67 = acc[:, 6:7] + acc[:, 7:8]
    return (p01 + p23) + (p45 + p67)


def _l2norm_rows(a):
    s = _rowsum64(a * a)
    n = jnp.sqrt(s)
    n = n + (s - n * n) / (2.0 * n)
    n = jnp.maximum(n, _EPS)
    q = a / n
    return q + (a - q * n) / n


def _lane_argmin(bd, bi):
    m = jnp.min(bd, axis=1, keepdims=True)
    cand = jnp.where(bd == m, bi, jnp.int32(_INT_MAX))
    return m, jnp.min(cand, axis=1, keepdims=True)


def _normalize_body(a_ref, o_ref):
    o_ref[...] = _l2norm_rows(a_ref[...]).astype(jnp.bfloat16)


def _normalize_bf16(a, tile):
    return pl.pallas_call(
        _normalize_body,
        grid=(a.shape[0] // tile,),
        in_specs=[pl.BlockSpec((tile, _D), lambda i: (i, 0))],
        out_specs=pl.BlockSpec((tile, _D), lambda i: (i, 0)),
        out_shape=jax.ShapeDtypeStruct(a.shape, jnp.bfloat16),
        compiler_params=pltpu.CompilerParams(
            dimension_semantics=("parallel",)),
    )(a)


def _argmin_body(x_ref, w_ref, out_ref, bd_ref, bi_ref, m1_ref, i1_ref):
    c = pl.program_id(1)

    t = lax.dot_general(x_ref[...], w_ref[...],
                        (((1,), (1,)), ((), ())),
                        preferred_element_type=jnp.float32)
    d = 2.0 - 2.0 * t

    half = pl.num_programs(1) // 2

    @pl.when((c == 0) | (c == half))
    def _():
        bd_ref[...] = jnp.full((_TM, _LANES), jnp.inf, jnp.float32)
        bi_ref[...] = jnp.zeros((_TM, _LANES), jnp.int32)

    lane = lax.broadcasted_iota(jnp.int32, (_TM, _LANES), 1)
    for k in range(_CB // _LANES):
        cols = c * _CB + k * _LANES + lane
        sk = d[:, k * _LANES:(k + 1) * _LANES]
        bd = bd_ref[...]
        upd = sk < bd
        bd_ref[...] = jnp.where(upd, sk, bd)
        bi_ref[...] = jnp.where(upd, cols, bi_ref[...])

    @pl.when(c == half - 1)
    def _():
        m1, i1 = _lane_argmin(bd_ref[...], bi_ref[...])
        m1_ref[...] = m1.astype(jnp.bfloat16).astype(jnp.float32)
        i1_ref[...] = i1

    @pl.when(c == pl.num_programs(1) - 1)
    def _():
        m2, i2 = _lane_argmin(bd_ref[...], bi_ref[...])
        take2 = m2 < m1_ref[...]
        out_ref[...] = jnp.where(take2, i2, i1_ref[...])


def _argmin_indices(x, W):
    return pl.pallas_call(
        _argmin_body,
        grid=(_N // _TM, _V // _CB),
        in_specs=[
            pl.BlockSpec((_TM, _D), lambda t, c: (t, 0)),
            pl.BlockSpec((_CB, _D), lambda t, c: (c, 0)),
        ],
        out_specs=pl.BlockSpec((_TM, 1), lambda t, c: (t, 0)),
        out_shape=jax.ShapeDtypeStruct((_N, 1), jnp.int32),
        scratch_shapes=[
            pltpu.VMEM((_TM, _LANES), jnp.float32),
            pltpu.VMEM((_TM, _LANES), jnp.int32),
            pltpu.VMEM((_TM, 1), jnp.float32),
            pltpu.VMEM((_TM, 1), jnp.int32),
        ],
        compiler_params=pltpu.CompilerParams(
            dimension_semantics=("parallel", "arbitrary")),
    )(x, W)


def _sc_gather_body(table_hbm, idx_hbm, out_hbm, idx_v, rows_v, sem):
    wid = lax.axis_index("s") * _NC + lax.axis_index("c")
    base = wid * _BW
    pltpu.sync_copy(idx_hbm.at[wid], idx_v)
    copies = []
    for g in range(_NG):
        copies.append(pltpu.async_copy(
            table_hbm.at[idx_v.at[g]],
            rows_v.at[pl.ds(g * _IG, _IG)],
            sem))
    for cp in copies:
        cp.wait()
    pltpu.sync_copy(rows_v, out_hbm.at[pl.ds(base, _BW)])


def _sc_gather(W, idx_groups):
    mesh = plsc.VectorSubcoreMesh(core_axis_name="c", subcore_axis_name="s")
    k = functools.partial(
        pl.kernel,
        out_type=jax.ShapeDtypeStruct((_N, _D), jnp.float32),
        scratch_types=[
            pltpu.VMEM((_NG, _IG), jnp.int32),
            pltpu.VMEM((_BW, _D), jnp.float32),
            pltpu.SemaphoreType.DMA,
        ],
        mesh=mesh,
        compiler_params=pltpu.CompilerParams(use_tc_tiling_on_sc=False),
    )(_sc_gather_body)
    return k(W, idx_groups)


def _st_loss_body(x_ref, q_ref, st_ref, loss_ref, acc_ref):
    i = pl.program_id(0)
    x = x_ref[...]
    q = q_ref[...]
    dqx = q - x
    st_ref[...] = x + dqx
    s = jnp.sum(dqx * dqx)

    @pl.when(i == 0)
    def _():
        acc_ref[0, 0] = 0.0

    acc_ref[0, 0] += s

    @pl.when(i == pl.num_programs(0) - 1)
    def _():
        loss_ref[...] = jnp.full(
            (1, 1), acc_ref[0, 0] * (1.0 / float(_N * _D)), jnp.float32)


def _st_loss(x, q):
    return pl.pallas_call(
        _st_loss_body,
        grid=(_N // _TM,),
        in_specs=[
            pl.BlockSpec((_TM, _D), lambda i: (i, 0)),
            pl.BlockSpec((_TM, _D), lambda i: (i, 0)),
        ],
        out_specs=[
            pl.BlockSpec((_TM, _D), lambda i: (i, 0)),
            pl.BlockSpec((1, 1), lambda i: (0, 0)),
        ],
        out_shape=[
            jax.ShapeDtypeStruct((_N, _D), jnp.float32),
            jax.ShapeDtypeStruct((1, 1), jnp.float32),
        ],
        scratch_shapes=[pltpu.SMEM((1, 1), jnp.float32)],
        compiler_params=pltpu.CompilerParams(
            dimension_semantics=("arbitrary",)),
    )(x, q)


def kernel(inputs, W):
    x = inputs.reshape(-1, _D)
    xn = _normalize_bf16(x, _TM)
    wn = _normalize_bf16(W, _CB)
    idx2d = _argmin_indices(xn, wn)
    idx_groups = idx2d.reshape(_NW, _NG, _IG)
    q = _sc_gather(W, idx_groups)
    st, loss = _st_loss(x, q)
    return st.reshape(inputs.shape), loss.reshape(()), idx2d


# speedup vs baseline: 5.2876x; 1.5591x over previous
import functools

import jax
import jax.numpy as jnp
from jax import lax
from jax.experimental import pallas as pl
from jax.experimental.pallas import tpu as pltpu
from jax.experimental.pallas import tpu_sc as plsc

_N = 18432
_D = 64
_V = 8192

_TM = 2048
_CB = 2048
_LANES = 128

_NC = 2
_NS = 16
_NW = _NC * _NS
_BW = _N // _NW
_IG = 96
_NG = _BW // _IG

_EPS = 1e-12
_INT_MAX = 2**31 - 1


def _rowsum64(a2):
    acc = a2[:, 0:8]
    for k in range(1, 8):
        acc = acc + a2[:, 8 * k:8 * k + 8]
    p01 = acc[:, 0:1] + acc[:, 1:2]
    p23 = acc[:, 2:3] + acc[:, 3:4]
    p45 = acc[:, 4:5] + acc[:, 5:6]
    p67 = acc[:, 6:7] + acc[:, 7:8]
    return (p01 + p23) + (p45 + p67)


def _l2norm_rows(a):
    s = _rowsum64(a * a)
    n = jnp.sqrt(s)
    n = n + (s - n * n) / (2.0 * n)
    n = jnp.maximum(n, _EPS)
    q = a / n
    return q + (a - q * n) / n


def _lane_argmin(bd, bi):
    m = jnp.min(bd, axis=1, keepdims=True)
    cand = jnp.where(bd == m, bi, jnp.int32(_INT_MAX))
    return m, jnp.min(cand, axis=1, keepdims=True)


def _normalize_body(a_ref, o_ref):
    o_ref[...] = _l2norm_rows(a_ref[...]).astype(jnp.bfloat16)


def _normalize_bf16(a, tile):
    return pl.pallas_call(
        _normalize_body,
        grid=(a.shape[0] // tile,),
        in_specs=[pl.BlockSpec((tile, _D), lambda i: (i, 0))],
        out_specs=pl.BlockSpec((tile, _D), lambda i: (i, 0)),
        out_shape=jax.ShapeDtypeStruct(a.shape, jnp.bfloat16),
        compiler_params=pltpu.CompilerParams(
            dimension_semantics=("parallel",)),
    )(a)


def _argmin_body(x_ref, w_ref, out_ref, bd_ref, bi_ref, m1_ref, i1_ref):
    c = pl.program_id(1)

    t = lax.dot_general(x_ref[...], w_ref[...],
                        (((1,), (1,)), ((), ())),
                        preferred_element_type=jnp.float32)
    d = 2.0 - 2.0 * t

    half = pl.num_programs(1) // 2
    reset = (c == 0) | (c == half)
    bd = jnp.where(reset, jnp.inf, bd_ref[...])
    bi = jnp.where(reset, 0, bi_ref[...])

    lane = lax.broadcasted_iota(jnp.int32, (_TM, _LANES), 1)
    for k in range(_CB // _LANES):
        cols = c * _CB + k * _LANES + lane
        sk = d[:, k * _LANES:(k + 1) * _LANES]
        upd = sk < bd
        bd = jnp.where(upd, sk, bd)
        bi = jnp.where(upd, cols, bi)
    bd_ref[...] = bd
    bi_ref[...] = bi

    @pl.when(c == half - 1)
    def _():
        m1, i1 = _lane_argmin(bd_ref[...], bi_ref[...])
        m1_ref[...] = m1.astype(jnp.bfloat16).astype(jnp.float32)
        i1_ref[...] = i1

    @pl.when(c == pl.num_programs(1) - 1)
    def _():
        m2, i2 = _lane_argmin(bd_ref[...], bi_ref[...])
        take2 = m2 < m1_ref[...]
        out_ref[...] = jnp.where(take2, i2, i1_ref[...])


def _argmin_indices(x, W):
    return pl.pallas_call(
        _argmin_body,
        grid=(_N // _TM, _V // _CB),
        in_specs=[
            pl.BlockSpec((_TM, _D), lambda t, c: (t, 0)),
            pl.BlockSpec((_CB, _D), lambda t, c: (c, 0)),
        ],
        out_specs=pl.BlockSpec((_TM, 1), lambda t, c: (t, 0)),
        out_shape=jax.ShapeDtypeStruct((_N, 1), jnp.int32),
        scratch_shapes=[
            pltpu.VMEM((_TM, _LANES), jnp.float32),
            pltpu.VMEM((_TM, _LANES), jnp.int32),
            pltpu.VMEM((_TM, 1), jnp.float32),
            pltpu.VMEM((_TM, 1), jnp.int32),
        ],
        compiler_params=pltpu.CompilerParams(
            dimension_semantics=("parallel", "arbitrary")),
    )(x, W)


def _sc_gather_body(table_hbm, idx_hbm, out_hbm, idx_v, rows_v, sem):
    wid = lax.axis_index("s") * _NC + lax.axis_index("c")
    base = wid * _BW
    pltpu.sync_copy(idx_hbm.at[wid], idx_v)
    copies = []
    for g in range(_NG):
        copies.append(pltpu.async_copy(
            table_hbm.at[idx_v.at[g]],
            rows_v.at[pl.ds(g * _IG, _IG)],
            sem))
    for cp in copies:
        cp.wait()
    pltpu.sync_copy(rows_v, out_hbm.at[pl.ds(base, _BW)])


def _sc_gather(W, idx_groups):
    mesh = plsc.VectorSubcoreMesh(core_axis_name="c", subcore_axis_name="s")
    k = functools.partial(
        pl.kernel,
        out_type=jax.ShapeDtypeStruct((_N, _D), jnp.float32),
        scratch_types=[
            pltpu.VMEM((_NG, _IG), jnp.int32),
            pltpu.VMEM((_BW, _D), jnp.float32),
            pltpu.SemaphoreType.DMA,
        ],
        mesh=mesh,
        compiler_params=pltpu.CompilerParams(use_tc_tiling_on_sc=False),
    )(_sc_gather_body)
    return k(W, idx_groups)


def _st_loss_body(x_ref, q_ref, st_ref, loss_ref, acc_ref):
    i = pl.program_id(0)
    x = x_ref[...]
    q = q_ref[...]
    dqx = q - x
    st_ref[...] = x + dqx
    s = jnp.sum(dqx * dqx)

    @pl.when(i == 0)
    def _():
        acc_ref[0, 0] = 0.0

    acc_ref[0, 0] += s

    @pl.when(i == pl.num_programs(0) - 1)
    def _():
        loss_ref[...] = jnp.full(
            (1, 1), acc_ref[0, 0] * (1.0 / float(_N * _D)), jnp.float32)


def _st_loss(x, q):
    return pl.pallas_call(
        _st_loss_body,
        grid=(_N // _TM,),
        in_specs=[
            pl.BlockSpec((_TM, _D), lambda i: (i, 0)),
            pl.BlockSpec((_TM, _D), lambda i: (i, 0)),
        ],
        out_specs=[
            pl.BlockSpec((_TM, _D), lambda i: (i, 0)),
            pl.BlockSpec((1, 1), lambda i: (0, 0)),
        ],
        out_shape=[
            jax.ShapeDtypeStruct((_N, _D), jnp.float32),
            jax.ShapeDtypeStruct((1, 1), jnp.float32),
        ],
        scratch_shapes=[pltpu.SMEM((1, 1), jnp.float32)],
        compiler_params=pltpu.CompilerParams(
            dimension_semantics=("arbitrary",)),
    )(x, q)


def kernel(inputs, W):
    x = inputs.reshape(-1, _D)
    xn = _normalize_bf16(x, _TM)
    wn = _normalize_bf16(W, _CB)
    idx2d = _argmin_indices(xn, wn)
    idx_groups = idx2d.reshape(_NW, _NG, _IG)
    q = _sc_gather(W, idx_groups)
    st, loss = _st_loss(x, q)
    return st.reshape(inputs.shape), loss.reshape(()), idx2d
